# baseline (speedup 1.0000x reference)
import functools

import jax
import jax.numpy as jnp
from jax import lax
from jax.experimental import pallas as pl
from jax.experimental.pallas import tpu as pltpu
from jax.experimental.pallas import tpu_sc as plsc

_EMB = 32
_BATCH = 16384
_CH = 128


def _sc_gather(u_table, s_table, u_idx3, s_idx3, num_cores, b_per_w):
    nch = b_per_w // _CH
    mesh = plsc.VectorSubcoreMesh(core_axis_name="c", subcore_axis_name="s")

    @functools.partial(
        pl.kernel,
        mesh=mesh,
        compiler_params=pltpu.CompilerParams(use_tc_tiling_on_sc=False),
        out_type=(
            jax.ShapeDtypeStruct((_BATCH, _EMB), jnp.float32),
            jax.ShapeDtypeStruct((_BATCH, _EMB), jnp.float32),
        ),
        scratch_types=[
            pltpu.VMEM((nch, _CH), jnp.int32),
            pltpu.VMEM((nch, _CH), jnp.int32),
            pltpu.VMEM((b_per_w, _EMB), jnp.float32),
            pltpu.VMEM((b_per_w, _EMB), jnp.float32),
            pltpu.SemaphoreType.DMA,
            pltpu.SemaphoreType.DMA,
        ],
    )
    def gather_k(u_tab, s_tab, u_ix, s_ix, u_out, s_out,
                 uidx_v, sidx_v, urows_v, srows_v, usem, ssem):
        wid = lax.axis_index("s") * num_cores + lax.axis_index("c")
        base = wid * b_per_w
        pltpu.sync_copy(u_ix.at[wid], uidx_v)
        pltpu.sync_copy(s_ix.at[wid], sidx_v)
        copies = []
        for c in range(nch):
            copies.append(pltpu.async_copy(
                u_tab.at[uidx_v.at[c]], urows_v.at[pl.ds(c * _CH, _CH)], usem))
            copies.append(pltpu.async_copy(
                s_tab.at[sidx_v.at[c]], srows_v.at[pl.ds(c * _CH, _CH)], ssem))
        for cp in copies:
            cp.wait()
        pltpu.sync_copy(urows_v, u_out.at[pl.ds(base, b_per_w)])
        pltpu.sync_copy(srows_v, s_out.at[pl.ds(base, b_per_w)])

    return gather_k(u_table, s_table, u_idx3, s_idx3)


def _mlp_kernel(u_ref, s_ref, w1_ref, b1_ref, w2_ref, b2_ref, o_ref):
    u = u_ref[...]
    s = s_ref[...]
    e = u * s
    w1 = w1_ref[...]
    acc = jnp.dot(e, w1[0:_EMB], preferred_element_type=jnp.float32)
    acc += jnp.dot(u, w1[_EMB:2 * _EMB], preferred_element_type=jnp.float32)
    acc += jnp.dot(s, w1[2 * _EMB:3 * _EMB], preferred_element_type=jnp.float32)
    h = jnp.maximum(acc + b1_ref[...], 0.0)
    logits = jnp.dot(h, w2_ref[...], preferred_element_type=jnp.float32)
    o_ref[...] = jax.nn.sigmoid(logits + b2_ref[...])


def _tc_mlp(u_emb, s_emb, W1, b1, W2, b2):
    bt = 4096
    grid = (_BATCH // bt,)
    return pl.pallas_call(
        _mlp_kernel,
        grid=grid,
        in_specs=[
            pl.BlockSpec((bt, _EMB), lambda i: (i, 0)),
            pl.BlockSpec((bt, _EMB), lambda i: (i, 0)),
            pl.BlockSpec((3 * _EMB, 8), lambda i: (0, 0)),
            pl.BlockSpec((1, 8), lambda i: (0, 0)),
            pl.BlockSpec((8, 1), lambda i: (0, 0)),
            pl.BlockSpec((1, 1), lambda i: (0, 0)),
        ],
        out_specs=pl.BlockSpec((bt, 1), lambda i: (i, 0)),
        out_shape=jax.ShapeDtypeStruct((_BATCH, 1), jnp.float32),
    )(u_emb, s_emb, W1, b1.reshape(1, 8), W2, b2.reshape(1, 1))


def kernel(mashup_inputs, user_inputs, service_inputs, user_table,
           service_table, W1, b1, W2, b2):
    info = plsc.get_sparse_core_info()
    n_workers = info.num_cores * info.num_subcores
    b_per_w = _BATCH // n_workers
    u_idx3 = user_inputs.reshape(n_workers, b_per_w // _CH, _CH)
    s_idx3 = service_inputs.reshape(n_workers, b_per_w // _CH, _CH)
    u_emb, s_emb = _sc_gather(user_table, service_table, u_idx3, s_idx3,
                              info.num_cores, b_per_w)
    return _tc_mlp(u_emb, s_emb, W1, b1, W2, b2)
